# Initial kernel scaffold; baseline (speedup 1.0000x reference)
import jax
import jax.numpy as jnp
from jax.experimental import pallas as pl


def kernel(x, latent):
    raise NotImplementedError("write your pallas kernel here")


# speedup vs baseline: 7.1204x; 7.1204x over previous
import functools

import jax
import jax.numpy as jnp
from jax import lax
from jax.experimental import pallas as pl
from jax.experimental.pallas import tpu as pltpu
from jax.experimental.pallas import tpu_sc as plsc

_MIX = 0.8
_K = 3
_BM = 128
_NC, _NS = 2, 16
_NW = _NC * _NS
_CH = 16


def _normalize_body(h_ref, out_ref):
    h = h_ref[...]
    norm = jnp.sqrt(jnp.sum(h * h, axis=1, keepdims=True))
    out_ref[...] = h / jnp.maximum(norm, 1e-12)


def _simtopk_body(hn_ref, rand_ref, idx_ref):
    i = pl.program_id(0)
    b = hn_ref.shape[0]
    lhs = hn_ref[pl.ds(i * _BM, _BM), :]
    sim = lax.dot_general(
        lhs, hn_ref[...], (((1,), (1,)), ((), ())),
        preferred_element_type=jnp.float32,
    )
    rowg = i * _BM + lax.broadcasted_iota(jnp.int32, (_BM, b), 0)
    colg = lax.broadcasted_iota(jnp.int32, (_BM, b), 1)
    sim = jnp.where(rowg == colg, jnp.float32(-9e15), sim)
    m1 = jnp.max(sim, axis=1, keepdims=True)
    s2 = jnp.where(sim == m1, -jnp.inf, sim)
    m2 = jnp.max(s2, axis=1, keepdims=True)
    s3 = jnp.where(s2 == m2, -jnp.inf, s2)
    m3 = jnp.max(s3, axis=1, keepdims=True)
    r = rand_ref[...]
    v = jnp.where(r == 0, m1, jnp.where(r == 1, m2, m3))
    cand = jnp.where(sim == v, colg, b)
    idx_ref[...] = jnp.min(cand, axis=1, keepdims=True)


def _mix_body(x_ref, xg_ref, out_ref):
    out_ref[...] = _MIX * x_ref[...] + (1.0 - _MIX) * xg_ref[...]


def _sc_gather(x_hbm, idx_hbm, out_hbm, idxc_v, rows_v, sem):
    bpw = idx_hbm.shape[0] // _NW
    wid = lax.axis_index("s") * _NC + lax.axis_index("c")
    base = wid * bpw

    def chunk(c, carry):
        cb = pl.multiple_of(base + c * _CH, 8)
        pltpu.sync_copy(idx_hbm.at[pl.ds(cb, _CH)], idxc_v)
        pltpu.async_copy(x_hbm.at[idxc_v], rows_v, sem).wait()
        pltpu.sync_copy(rows_v, out_hbm.at[pl.ds(cb, _CH)])
        return carry

    lax.fori_loop(0, bpw // _CH, chunk, 0)


def kernel(x, latent):
    b, d = x.shape

    hn = pl.pallas_call(
        _normalize_body,
        out_shape=jax.ShapeDtypeStruct(latent.shape, jnp.float32),
    )(latent)

    rkey = jax.random.fold_in(jax.random.key(0), 123)
    rand_idx = jax.random.randint(rkey, (b,), 0, _K).astype(jnp.int32)

    idx2d = pl.pallas_call(
        _simtopk_body,
        grid=(b // _BM,),
        in_specs=[
            pl.BlockSpec(latent.shape, lambda i: (0, 0)),
            pl.BlockSpec((_BM, 1), lambda i: (i, 0)),
        ],
        out_specs=pl.BlockSpec((_BM, 1), lambda i: (i, 0)),
        out_shape=jax.ShapeDtypeStruct((b, 1), jnp.int32),
        compiler_params=pltpu.CompilerParams(
            dimension_semantics=("arbitrary",),
        ),
    )(hn, rand_idx.reshape(b, 1))
    nbr = idx2d.reshape(b)

    gather = pl.kernel(
        _sc_gather,
        out_type=jax.ShapeDtypeStruct((b, d), jnp.float32),
        mesh=plsc.VectorSubcoreMesh(
            core_axis_name="c", subcore_axis_name="s",
            num_cores=_NC, num_subcores=_NS,
        ),
        scratch_types=[
            pltpu.VMEM((_CH,), jnp.int32),
            pltpu.VMEM((_CH, d), jnp.float32),
            pltpu.SemaphoreType.DMA,
        ],
    )
    xg = gather(x, nbr)

    out = pl.pallas_call(
        _mix_body,
        grid=(b // 256,),
        in_specs=[
            pl.BlockSpec((256, d), lambda i: (i, 0)),
            pl.BlockSpec((256, d), lambda i: (i, 0)),
        ],
        out_specs=pl.BlockSpec((256, d), lambda i: (i, 0)),
        out_shape=jax.ShapeDtypeStruct((b, d), jnp.float32),
    )(x, xg)
    return out


# speedup vs baseline: 7.6179x; 1.0699x over previous
import functools

import jax
import jax.numpy as jnp
from jax import lax
from jax.experimental import pallas as pl
from jax.experimental.pallas import tpu as pltpu
from jax.experimental.pallas import tpu_sc as plsc

_MIX = 0.8
_K = 3
_BM = 128
_NC, _NS = 2, 16
_NW = _NC * _NS
_CH = 16


def _simtopk_body(lat_ref, rand_ref, idx_ref, hn_ref):
    i = pl.program_id(0)
    b = lat_ref.shape[0]

    @pl.when(i == 0)
    def _():
        h = lat_ref[...]
        norm = jnp.sqrt(jnp.sum(h * h, axis=1, keepdims=True))
        hn_ref[...] = h / jnp.maximum(norm, 1e-12)

    lhs = hn_ref[pl.ds(i * _BM, _BM), :]
    sim = lax.dot_general(
        lhs, hn_ref[...], (((1,), (1,)), ((), ())),
        preferred_element_type=jnp.float32,
    )
    rowg = i * _BM + lax.broadcasted_iota(jnp.int32, (_BM, b), 0)
    colg = lax.broadcasted_iota(jnp.int32, (_BM, b), 1)
    sim = jnp.where(rowg == colg, jnp.float32(-9e15), sim)

    nt = b // 128
    m1 = sim[:, 0:128]
    ninf = jnp.full((_BM, 128), -jnp.inf, jnp.float32)
    m2 = ninf
    m3 = ninf
    for q in range(1, nt):
        t = sim[:, q * 128:(q + 1) * 128]
        lo1 = jnp.minimum(m1, t)
        m1 = jnp.maximum(m1, t)
        lo2 = jnp.minimum(m2, lo1)
        m2 = jnp.maximum(m2, lo1)
        m3 = jnp.maximum(m3, lo2)
    cat = jnp.concatenate([m1, m2, m3], axis=1)
    v1 = jnp.max(cat, axis=1, keepdims=True)
    c2 = jnp.where(cat == v1, -jnp.inf, cat)
    v2 = jnp.max(c2, axis=1, keepdims=True)
    c3 = jnp.where(c2 == v2, -jnp.inf, c2)
    v3 = jnp.max(c3, axis=1, keepdims=True)
    r = rand_ref[...]
    v = jnp.where(r == 0, v1, jnp.where(r == 1, v2, v3))
    cand = jnp.where(sim == v, colg, b)
    idx_ref[...] = jnp.min(cand, axis=1, keepdims=True)


def _mix_body(x_ref, xg_ref, out_ref):
    out_ref[...] = _MIX * x_ref[...] + (1.0 - _MIX) * xg_ref[...]


def _sc_gather(x_hbm, idx_hbm, out_hbm, idxc_v, rows_v, sem):
    bpw = idx_hbm.shape[0] // _NW
    wid = lax.axis_index("s") * _NC + lax.axis_index("c")
    base = wid * bpw

    def chunk(c, carry):
        cb = pl.multiple_of(base + c * _CH, 8)
        pltpu.sync_copy(idx_hbm.at[pl.ds(cb, _CH)], idxc_v)
        pltpu.async_copy(x_hbm.at[idxc_v], rows_v, sem).wait()
        pltpu.sync_copy(rows_v, out_hbm.at[pl.ds(cb, _CH)])
        return carry

    lax.fori_loop(0, bpw // _CH, chunk, 0)


def kernel(x, latent):
    b, d = x.shape

    rkey = jax.random.fold_in(jax.random.key(0), 123)
    rand_idx = jax.random.randint(rkey, (b,), 0, _K).astype(jnp.int32)

    idx2d = pl.pallas_call(
        _simtopk_body,
        grid=(b // _BM,),
        in_specs=[
            pl.BlockSpec(latent.shape, lambda i: (0, 0)),
            pl.BlockSpec((_BM, 1), lambda i: (i, 0)),
        ],
        out_specs=pl.BlockSpec((_BM, 1), lambda i: (i, 0)),
        out_shape=jax.ShapeDtypeStruct((b, 1), jnp.int32),
        scratch_shapes=[pltpu.VMEM(latent.shape, jnp.float32)],
        compiler_params=pltpu.CompilerParams(
            dimension_semantics=("arbitrary",),
        ),
    )(latent, rand_idx.reshape(b, 1))
    nbr = idx2d.reshape(b)

    gather = pl.kernel(
        _sc_gather,
        out_type=jax.ShapeDtypeStruct((b, d), jnp.float32),
        mesh=plsc.VectorSubcoreMesh(
            core_axis_name="c", subcore_axis_name="s",
            num_cores=_NC, num_subcores=_NS,
        ),
        scratch_types=[
            pltpu.VMEM((_CH,), jnp.int32),
            pltpu.VMEM((_CH, d), jnp.float32),
            pltpu.SemaphoreType.DMA,
        ],
    )
    xg = gather(x, nbr)

    out = pl.pallas_call(
        _mix_body,
        grid=(b // 256,),
        in_specs=[
            pl.BlockSpec((256, d), lambda i: (i, 0)),
            pl.BlockSpec((256, d), lambda i: (i, 0)),
        ],
        out_specs=pl.BlockSpec((256, d), lambda i: (i, 0)),
        out_shape=jax.ShapeDtypeStruct((b, d), jnp.float32),
    )(x, xg)
    return out


# speedup vs baseline: 8.4235x; 1.1057x over previous
import functools

import jax
import jax.numpy as jnp
from jax import lax
from jax.experimental import pallas as pl
from jax.experimental.pallas import tpu as pltpu
from jax.experimental.pallas import tpu_sc as plsc

_MIX = 0.8
_K = 3
_BM = 256
_NC, _NS = 2, 16
_NW = _NC * _NS
_CH = 16


def _simtopk_body(lat_ref, rand_ref, idx_ref, hn_ref):
    i = pl.program_id(0)
    b = lat_ref.shape[0]

    @pl.when(i == 0)
    def _():
        h = lat_ref[...]
        norm = jnp.sqrt(jnp.sum(h * h, axis=1, keepdims=True))
        hn_ref[...] = h / jnp.maximum(norm, 1e-12)

    lhs = hn_ref[pl.ds(i * _BM, _BM), :]
    sim = lax.dot_general(
        lhs, hn_ref[...], (((1,), (1,)), ((), ())),
        preferred_element_type=jnp.float32,
    )
    rowg = i * _BM + lax.broadcasted_iota(jnp.int32, (_BM, b), 0)
    colg = lax.broadcasted_iota(jnp.int32, (_BM, b), 1)
    sim = jnp.where(rowg == colg, jnp.float32(-9e15), sim)

    nt = b // 128
    m1 = sim[:, 0:128]
    ninf = jnp.full((_BM, 128), -jnp.inf, jnp.float32)
    m2 = ninf
    m3 = ninf
    for q in range(1, nt):
        t = sim[:, q * 128:(q + 1) * 128]
        lo1 = jnp.minimum(m1, t)
        m1 = jnp.maximum(m1, t)
        lo2 = jnp.minimum(m2, lo1)
        m2 = jnp.maximum(m2, lo1)
        m3 = jnp.maximum(m3, lo2)
    cat = jnp.concatenate([m1, m2, m3], axis=1)
    v1 = jnp.max(cat, axis=1, keepdims=True)
    c2 = jnp.where(cat == v1, -jnp.inf, cat)
    v2 = jnp.max(c2, axis=1, keepdims=True)
    c3 = jnp.where(c2 == v2, -jnp.inf, c2)
    v3 = jnp.max(c3, axis=1, keepdims=True)
    r = rand_ref[...]
    v = jnp.where(r == 0, v1, jnp.where(r == 1, v2, v3))
    cand = jnp.where(sim == v, colg, b)
    idx_ref[...] = jnp.min(cand, axis=1, keepdims=True)


def _mix_body(x_ref, xg_ref, out_ref):
    out_ref[...] = _MIX * x_ref[...] + (1.0 - _MIX) * xg_ref[...]


def _sc_gather(x_hbm, idx_hbm, out_hbm, idxc_v, rows_v, sem):
    bpw = idx_hbm.shape[0] // _NW
    wid = lax.axis_index("s") * _NC + lax.axis_index("c")
    base = wid * bpw

    def chunk(c, carry):
        cb = pl.multiple_of(base + c * _CH, 8)
        pltpu.sync_copy(idx_hbm.at[pl.ds(cb, _CH)], idxc_v)
        pltpu.async_copy(x_hbm.at[idxc_v], rows_v, sem).wait()
        pltpu.sync_copy(rows_v, out_hbm.at[pl.ds(cb, _CH)])
        return carry

    lax.fori_loop(0, bpw // _CH, chunk, 0)


def kernel(x, latent):
    b, d = x.shape

    rkey = jax.random.fold_in(jax.random.key(0), 123)
    rand_idx = jax.random.randint(rkey, (b,), 0, _K).astype(jnp.int32)

    idx2d = pl.pallas_call(
        _simtopk_body,
        grid=(b // _BM,),
        in_specs=[
            pl.BlockSpec(latent.shape, lambda i: (0, 0)),
            pl.BlockSpec((_BM, 1), lambda i: (i, 0)),
        ],
        out_specs=pl.BlockSpec((_BM, 1), lambda i: (i, 0)),
        out_shape=jax.ShapeDtypeStruct((b, 1), jnp.int32),
        scratch_shapes=[pltpu.VMEM(latent.shape, jnp.float32)],
        compiler_params=pltpu.CompilerParams(
            dimension_semantics=("arbitrary",),
        ),
    )(latent, rand_idx.reshape(b, 1))
    nbr = idx2d.reshape(b)

    gather = pl.kernel(
        _sc_gather,
        out_type=jax.ShapeDtypeStruct((b, d), jnp.float32),
        mesh=plsc.VectorSubcoreMesh(
            core_axis_name="c", subcore_axis_name="s",
            num_cores=_NC, num_subcores=_NS,
        ),
        scratch_types=[
            pltpu.VMEM((_CH,), jnp.int32),
            pltpu.VMEM((_CH, d), jnp.float32),
            pltpu.SemaphoreType.DMA,
        ],
    )
    xg = gather(x, nbr)

    out = pl.pallas_call(
        _mix_body,
        grid=(b // 256,),
        in_specs=[
            pl.BlockSpec((256, d), lambda i: (i, 0)),
            pl.BlockSpec((256, d), lambda i: (i, 0)),
        ],
        out_specs=pl.BlockSpec((256, d), lambda i: (i, 0)),
        out_shape=jax.ShapeDtypeStruct((b, d), jnp.float32),
    )(x, xg)
    return out


# speedup vs baseline: 8.8480x; 1.0504x over previous
import functools

import jax
import jax.numpy as jnp
from jax import lax
from jax.experimental import pallas as pl
from jax.experimental.pallas import tpu as pltpu
from jax.experimental.pallas import tpu_sc as plsc

_MIX = 0.8
_K = 3
_BM = 512
_NC, _NS = 2, 16
_NW = _NC * _NS
_CH = 16


def _simtopk_body(lat_ref, rand_ref, idx_ref, hn_ref):
    i = pl.program_id(0)
    b = lat_ref.shape[0]

    @pl.when(i == 0)
    def _():
        h = lat_ref[...]
        norm = jnp.sqrt(jnp.sum(h * h, axis=1, keepdims=True))
        hn_ref[...] = h / jnp.maximum(norm, 1e-12)

    lhs = hn_ref[pl.ds(i * _BM, _BM), :]
    sim = lax.dot_general(
        lhs, hn_ref[...], (((1,), (1,)), ((), ())),
        preferred_element_type=jnp.float32,
    )
    rowg = i * _BM + lax.broadcasted_iota(jnp.int32, (_BM, b), 0)
    colg = lax.broadcasted_iota(jnp.int32, (_BM, b), 1)
    sim = jnp.where(rowg == colg, jnp.float32(-9e15), sim)

    nt = b // 128
    m1 = sim[:, 0:128]
    ninf = jnp.full((_BM, 128), -jnp.inf, jnp.float32)
    m2 = ninf
    m3 = ninf
    for q in range(1, nt):
        t = sim[:, q * 128:(q + 1) * 128]
        lo1 = jnp.minimum(m1, t)
        m1 = jnp.maximum(m1, t)
        lo2 = jnp.minimum(m2, lo1)
        m2 = jnp.maximum(m2, lo1)
        m3 = jnp.maximum(m3, lo2)
    cat = jnp.concatenate([m1, m2, m3], axis=1)
    v1 = jnp.max(cat, axis=1, keepdims=True)
    c2 = jnp.where(cat == v1, -jnp.inf, cat)
    v2 = jnp.max(c2, axis=1, keepdims=True)
    c3 = jnp.where(c2 == v2, -jnp.inf, c2)
    v3 = jnp.max(c3, axis=1, keepdims=True)
    r = rand_ref[...]
    v = jnp.where(r == 0, v1, jnp.where(r == 1, v2, v3))
    cand = jnp.where(sim == v, colg, b)
    idx_ref[...] = jnp.min(cand, axis=1, keepdims=True)


def _mix_body(x_ref, xg_ref, out_ref):
    out_ref[...] = _MIX * x_ref[...] + (1.0 - _MIX) * xg_ref[...]


def _sc_gather(x_hbm, idx_hbm, out_hbm, idxc_v, rows_v, sem):
    bpw = idx_hbm.shape[0] // _NW
    wid = lax.axis_index("s") * _NC + lax.axis_index("c")
    base = wid * bpw

    def chunk(c, carry):
        cb = pl.multiple_of(base + c * _CH, 8)
        pltpu.sync_copy(idx_hbm.at[pl.ds(cb, _CH)], idxc_v)
        pltpu.async_copy(x_hbm.at[idxc_v], rows_v, sem).wait()
        pltpu.sync_copy(rows_v, out_hbm.at[pl.ds(cb, _CH)])
        return carry

    lax.fori_loop(0, bpw // _CH, chunk, 0)


def kernel(x, latent):
    b, d = x.shape

    rkey = jax.random.fold_in(jax.random.key(0), 123)
    rand_idx = jax.random.randint(rkey, (b,), 0, _K).astype(jnp.int32)

    idx2d = pl.pallas_call(
        _simtopk_body,
        grid=(b // _BM,),
        in_specs=[
            pl.BlockSpec(latent.shape, lambda i: (0, 0)),
            pl.BlockSpec((_BM, 1), lambda i: (i, 0)),
        ],
        out_specs=pl.BlockSpec((_BM, 1), lambda i: (i, 0)),
        out_shape=jax.ShapeDtypeStruct((b, 1), jnp.int32),
        scratch_shapes=[pltpu.VMEM(latent.shape, jnp.float32)],
        compiler_params=pltpu.CompilerParams(
            dimension_semantics=("arbitrary",),
        ),
    )(latent, rand_idx.reshape(b, 1))
    nbr = idx2d.reshape(b)

    gather = pl.kernel(
        _sc_gather,
        out_type=jax.ShapeDtypeStruct((b, d), jnp.float32),
        mesh=plsc.VectorSubcoreMesh(
            core_axis_name="c", subcore_axis_name="s",
            num_cores=_NC, num_subcores=_NS,
        ),
        scratch_types=[
            pltpu.VMEM((_CH,), jnp.int32),
            pltpu.VMEM((_CH, d), jnp.float32),
            pltpu.SemaphoreType.DMA,
        ],
    )
    xg = gather(x, nbr)

    out = pl.pallas_call(
        _mix_body,
        grid=(b // 256,),
        in_specs=[
            pl.BlockSpec((256, d), lambda i: (i, 0)),
            pl.BlockSpec((256, d), lambda i: (i, 0)),
        ],
        out_specs=pl.BlockSpec((256, d), lambda i: (i, 0)),
        out_shape=jax.ShapeDtypeStruct((b, d), jnp.float32),
    )(x, xg)
    return out
